# baseline (speedup 1.0000x reference)
import functools

import jax
import jax.numpy as jnp
from jax import lax
from jax.experimental import pallas as pl
from jax.experimental.pallas import tpu as pltpu
from jax.experimental.pallas import tpu_sc as plsc

N = 10000
NP = 10240
NP2 = NP // 5
D = 128
H = 128
C = 10
G = 64
NC = 2
NS = 16
NW = NC * NS
CH = 128
BR = 1024
SLOPE = 0.2
DOT = (((1,), (1,)), ((), ()))
PREC = lax.Precision.HIGHEST


def _leaky(t):
    return jnp.maximum(t, SLOPE * t)


def _elu(t):
    return jnp.where(t > 0, t, jnp.exp(t) - 1.0)


def _xform_tail(h, w_ref, atts_ref, attd_ref, xp_ref, as_ref, ad_ref,
                mx_ref, acc_ref):
    i = pl.program_id(0)
    xp = lax.dot_general(h, w_ref[...], DOT, precision=PREC,
                         preferred_element_type=jnp.float32)
    xp_ref[...] = xp
    a_s = lax.dot_general(xp, atts_ref[...], (((1,), (0,)), ((), ())),
                          precision=PREC, preferred_element_type=jnp.float32)
    a_d = lax.dot_general(xp, attd_ref[...], (((1,), (0,)), ((), ())),
                          precision=PREC, preferred_element_type=jnp.float32)
    as_ref[...] = a_s
    ad_ref[...] = a_d
    m = jnp.max(a_s)
    prev = jnp.where(i == 0, -jnp.inf, acc_ref[0, 0])
    cur = jnp.maximum(prev, m)
    acc_ref[0, 0] = cur

    @pl.when(i == pl.num_programs(0) - 1)
    def _():
        mx_ref[...] = jnp.full((1, 16), cur, jnp.float32)


def _xform_a_body(x_ref, w_ref, atts_ref, attd_ref,
                  xp_ref, as_ref, ad_ref, mx_ref, acc_ref):
    _xform_tail(x_ref[...], w_ref, atts_ref, attd_ref,
                xp_ref, as_ref, ad_ref, mx_ref, acc_ref)


def _combine(p_ref, wp_ref, b_ref):
    wsum = jnp.sum(wp_ref[...], axis=0) + 1e-16
    num = jnp.sum(p_ref[...], axis=0)
    return _elu(num / wsum[:, None] + b_ref[...])


def _xform_b_body(p_ref, wp_ref, b_ref, w_ref, atts_ref, attd_ref,
                  xp_ref, as_ref, ad_ref, mx_ref, acc_ref):
    h = _combine(p_ref, wp_ref, b_ref)
    _xform_tail(h, w_ref, atts_ref, attd_ref,
                xp_ref, as_ref, ad_ref, mx_ref, acc_ref)


def _final_body(p_ref, wp_ref, b_ref, mw1_ref, mb1_ref,
                mw2_ref, mb2_ref, batch_ref, out_ref):
    i = pl.program_id(0)
    h = _combine(p_ref, wp_ref, b_ref)
    z = lax.dot_general(h, mw1_ref[...], DOT, precision=PREC,
                        preferred_element_type=jnp.float32) + mb1_ref[...]
    z = jnp.maximum(z, 0.0)
    z = lax.dot_general(z, mw2_ref[...], DOT, precision=PREC,
                        preferred_element_type=jnp.float32) + mb2_ref[...]
    seg = lax.broadcasted_iota(jnp.int32, (BR, G), 1)
    oh = (batch_ref[...] == seg).astype(jnp.float32)
    contrib = lax.dot_general(oh, z, (((0,), (0,)), ((), ())),
                              precision=PREC,
                              preferred_element_type=jnp.float32)

    @pl.when(i == 0)
    def _():
        out_ref[...] = jnp.zeros_like(out_ref)

    out_ref[...] = out_ref[...] + contrib


def _row_spec(shape=(BR, H)):
    return pl.BlockSpec(shape, lambda i: (i, 0))


def _full(shape):
    return pl.BlockSpec(shape, lambda i: tuple(0 for _ in shape))


_PART_SPEC = pl.BlockSpec((NC, BR, H), lambda i: (0, i, 0))
_WP_SPEC = pl.BlockSpec((NW, BR), lambda i: (0, i))

_XFORM_OUTS = (
    jax.ShapeDtypeStruct((NP, H), jnp.float32),
    jax.ShapeDtypeStruct((NP, 1), jnp.float32),
    jax.ShapeDtypeStruct((NP, 1), jnp.float32),
    jax.ShapeDtypeStruct((1, 16), jnp.float32),
)
_XFORM_OUT_SPECS = [
    _row_spec(),
    _row_spec((BR, 1)),
    _row_spec((BR, 1)),
    _full((1, 16)),
]

_xform_a = pl.pallas_call(
    _xform_a_body,
    grid=(NP // BR,),
    in_specs=[
        _row_spec(),
        _full((H, D)),
        _full((D, 1)),
        _full((D, 1)),
    ],
    out_specs=_XFORM_OUT_SPECS,
    out_shape=_XFORM_OUTS,
    scratch_shapes=[pltpu.SMEM((1, 1), jnp.float32)],
)

_xform_b = pl.pallas_call(
    _xform_b_body,
    grid=(NP // BR,),
    in_specs=[
        _PART_SPEC,
        _WP_SPEC,
        _full((1, H)),
        _full((H, H)),
        _full((H, 1)),
        _full((H, 1)),
    ],
    out_specs=_XFORM_OUT_SPECS,
    out_shape=_XFORM_OUTS,
    scratch_shapes=[pltpu.SMEM((1, 1), jnp.float32)],
)

_final = pl.pallas_call(
    _final_body,
    grid=(NP // BR,),
    in_specs=[
        _PART_SPEC,
        _WP_SPEC,
        _full((1, H)),
        _full((H, H)),
        _full((1, H)),
        _full((C, H)),
        _full((1, C)),
        _row_spec((BR, 1)),
    ],
    out_specs=_full((G, C)),
    out_shape=jax.ShapeDtypeStruct((G, C), jnp.float32),
)


def _make_sc_edge(ncz, e_total):
    epw = ncz * CH
    lsz = epw + 4 * CH
    nrp = NP2 // NS
    npass = NP // NP2
    mesh = plsc.VectorSubcoreMesh(core_axis_name="c", subcore_axis_name="s",
                                  num_cores=NC, num_subcores=NS)

    @functools.partial(
        pl.kernel,
        out_type=(
            pltpu.MemorySpace.HBM((NC, NP, H), jnp.float32),
            pltpu.MemorySpace.HBM((NW, NP), jnp.float32),
        ),
        mesh=mesh,
        scratch_types=[
            pltpu.VMEM((ncz, CH), jnp.int32),
            pltpu.VMEM((lsz,), jnp.int32),
            pltpu.VMEM((lsz,), jnp.float32),
            pltpu.VMEM((3, CH), jnp.int32),
            pltpu.VMEM((3, CH), jnp.int32),
            pltpu.VMEM((3, CH), jnp.float32),
            pltpu.VMEM((NP,), jnp.float32),
            pltpu.VMEM((NP,), jnp.float32),
            pltpu.VMEM((1, 16), jnp.float32),
            pltpu.VMEM((NP,), jnp.float32),
            pltpu.VMEM((CH, H), jnp.float32),
            pltpu.VMEM((CH, H), jnp.float32),
            pltpu.VMEM((CH, H), jnp.float32),
            pltpu.VMEM_SHARED((NP2, H), jnp.float32),
            pltpu.SemaphoreType.DMA,
            pltpu.SemaphoreType.DMA,
            pltpu.SemaphoreType.DMA,
            pltpu.SemaphoreType.DMA,
            pltpu.SemaphoreType.DMA,
            pltpu.SemaphoreType.DMA,
        ],
        compiler_params=pltpu.CompilerParams(needs_layout_passes=False),
    )
    def sc_edge(pk_hbm, as_hbm, ad_hbm, mx_hbm, xp_hbm,
                out_hbm, wp_hbm,
                pk_v, plist, elist, sstage, dstp_v, exp_v, as_v, ad_v,
                mx_v, w_v, rows0_v, rows1_v, rows2_v, out_sp,
                g0, g1, g2, s0, s1, s2):
        rows = [rows0_v, rows1_v, rows2_v]
        gsem = [g0, g1, g2]
        ssem = [s0, s1, s2]
        rows_v = rows0_v
        cid = lax.axis_index("c")
        sid = lax.axis_index("s")
        wid = cid * NS + sid
        pltpu.sync_copy(pk_hbm.at[wid], pk_v)
        pltpu.sync_copy(as_hbm, as_v)
        pltpu.sync_copy(ad_hbm, ad_v)
        pltpu.sync_copy(mx_hbm, mx_v)

        zero16 = jnp.zeros((16,), jnp.float32)

        def _zw(i, carry):
            w_v[pl.ds(i * 16, 16)] = zero16
            return carry

        lax.fori_loop(0, NP // 16, _zw, 0)

        def _zr(i, carry):
            for j in range(H // 16):
                rows_v[i, pl.ds(j * 16, 16)] = zero16
            return carry

        def _zero_my_outsp_slice():
            for k in range(nrp // CH):
                pltpu.sync_copy(rows_v,
                                out_sp.at[pl.ds(sid * nrp + k * CH, CH)])
            rem = nrp % CH
            if rem:
                pltpu.sync_copy(
                    rows_v.at[pl.ds(0, rem)],
                    out_sp.at[pl.ds(sid * nrp + (nrp // CH) * CH, rem)])

        lax.fori_loop(0, CH, _zr, 0)
        _zero_my_outsp_slice()
        plsc.subcore_barrier()

        mxv = mx_v[0, :]
        ebase = wid * epw
        iota16 = lax.iota(jnp.int32, 16)
        zero_i = jnp.int32(0)

        def _unpack(c, g):
            sl = pl.ds(g * 16, 16)
            v16 = pk_v[c, sl]
            s16 = v16 & 0x3FFF
            d16 = lax.shift_right_logical(v16, 14)
            valid = (ebase + c * CH + g * 16 + iota16) < e_total
            return v16, s16, d16, valid

        def _count(c, cnts):
            def grp(g, cn):
                _, _, d16, valid = _unpack(c, g)
                new = []
                for p in range(npass):
                    m = valid & (d16 >= p * NP2) & (d16 < (p + 1) * NP2)
                    pc = plsc.all_reduce_population_count(m)[0]
                    new.append(cn[p] + pc)
                return tuple(new)

            return lax.fori_loop(0, CH // 16, grp, cnts)

        cnts = lax.fori_loop(0, ncz, _count,
                             tuple(zero_i for _ in range(npass)))
        offs = []
        acc = zero_i
        for p in range(npass):
            offs.append(acc)
            acc = acc + cnts[p]

        def _fill(c, ns):
            def grp(g, n):
                v16, s16, d16, valid = _unpack(c, g)
                a1 = plsc.load_gather(as_v, [s16])
                a2 = plsc.load_gather(ad_v, [d16])
                alpha = _leaky(a1 + a2)
                mb = _leaky(mxv + a2)
                ex = jnp.exp(alpha - mb)
                ex = jnp.where(valid, ex, 0.0)
                plsc.addupdate_scatter(w_v, [d16], ex)
                new = []
                for p in range(npass):
                    m = valid & (d16 >= p * NP2) & (d16 < (p + 1) * NP2)
                    at = offs[p] + n[p]
                    plsc.store_compressed(plist.at[pl.ds(at, 16)], v16, mask=m)
                    plsc.store_compressed(elist.at[pl.ds(at, 16)], ex, mask=m)
                    pc = plsc.all_reduce_population_count(m)[0]
                    new.append(n[p] + pc)
                return tuple(new)

            return lax.fori_loop(0, CH // 16, grp, ns)

        lax.fori_loop(0, ncz, _fill, tuple(zero_i for _ in range(npass)))

        def _stage(c, k, off, cnt, lo):
            base = off + c * CH
            for g in range(CH // 16):
                sl = pl.ds(g * 16, 16)
                v16 = plist[pl.ds(base + g * 16, 16)]
                e16 = elist[pl.ds(base + g * 16, 16)]
                m = (c * CH + g * 16 + iota16) < cnt
                s16 = jnp.where(m, v16 & 0x3FFF, 0)
                rel = jnp.where(m, lax.shift_right_logical(v16, 14) - lo, 0)
                sstage[k, sl] = s16
                dstp_v[k, sl] = rel
                exp_v[k, sl] = jnp.where(m, e16, 0.0)

        def _gather_start(k):
            pltpu.async_copy(xp_hbm.at[sstage.at[k]], rows[k], gsem[k])

        def _gather_wait(k):
            pltpu.make_async_copy(xp_hbm.at[sstage.at[k]], rows[k],
                                  gsem[k]).wait()

        def _scat_start(k):
            pltpu.async_copy(rows[k], out_sp.at[dstp_v.at[k]], ssem[k],
                             add=True)

        def _scat_wait(k):
            pltpu.make_async_copy(rows[k], out_sp.at[dstp_v.at[k]],
                                  ssem[k]).wait()

        def _scale(k):
            def body(g2, carry2):
                ex16 = exp_v[k, pl.ds(g2 * 16, 16)]
                for l in range(16):
                    sc = ex16[l]
                    for j in range(H // 16):
                        sl2 = pl.ds(j * 16, 16)
                        i = g2 * 16 + l
                        rows[k][i, sl2] = rows[k][i, sl2] * sc
                return carry2

            lax.fori_loop(0, CH // 16, body, 0)

        def _flush(row0):
            pltpu.sync_copy(out_sp.at[pl.ds(sid * nrp, nrp)],
                            out_hbm.at[cid, pl.ds(row0 + sid * nrp, nrp)])

        for p in range(npass):
            lo = p * NP2
            off = offs[p]
            cnt = cnts[p]
            n3 = jnp.maximum((cnt + 3 * CH - 1) // (3 * CH), 1)
            rc = 3 * n3
            if p:
                plsc.subcore_barrier()
                lax.fori_loop(0, CH, _zr, 0)
                _zero_my_outsp_slice()
                plsc.subcore_barrier()

            _stage(zero_i, 0, off, cnt, lo)
            _gather_start(0)

            def _tri(i3, carry, _off=off, _cnt=cnt, _lo=lo, _rc=rc):
                for k in range(3):
                    c = 3 * i3 + k
                    nxt = c + 1
                    kn = (k + 1) % 3

                    @pl.when(nxt < _rc)
                    def _():
                        @pl.when(c >= 2)
                        def _():
                            _scat_wait(kn)

                        _stage(nxt, kn, _off, _cnt, _lo)
                        _gather_start(kn)

                    _gather_wait(k)
                    _scale(k)
                    _scat_start(k)
                return carry

            lax.fori_loop(0, n3, _tri, 0)
            for k in range(3):
                _scat_wait(k)
            plsc.subcore_barrier()
            _flush(lo)
        pltpu.sync_copy(w_v, wp_hbm.at[wid])

    return sc_edge


def kernel(x, edge_index, batch, W0, att_src0, att_dst0, b0,
           W1, att_src1, att_dst1, b1, mlp_W1, mlp_b1, mlp_W2, mlp_b2):
    n = x.shape[0]
    e = edge_index.shape[1]
    e_total = e + n
    ncz = -(-e_total // (NW * CH))
    ncz += (-ncz) % 3
    e_pad = NW * ncz * CH

    loops = jnp.arange(n, dtype=edge_index.dtype)
    src = jnp.concatenate([edge_index[0], loops])
    dst = jnp.concatenate([edge_index[1], loops])
    packed = src | (dst << 14)
    pk_p = jnp.pad(packed, (0, e_pad - e_total)).reshape(NW, ncz, CH)
    x_p = jnp.pad(x, ((0, NP - n), (0, 0)))
    batch_p = jnp.pad(batch, (0, NP - n), constant_values=G).reshape(NP, 1)

    sc_edge = _make_sc_edge(ncz, e_total)

    xp0, as0, ad0, mx0 = _xform_a(
        x_p, W0, att_src0.reshape(D, 1), att_dst0.reshape(D, 1))
    o0, wp0 = sc_edge(pk_p, as0.reshape(NP), ad0.reshape(NP), mx0, xp0)
    xp1, as1, ad1, mx1 = _xform_b(
        o0, wp0, b0.reshape(1, H), W1,
        att_src1.reshape(H, 1), att_dst1.reshape(H, 1))
    o1, wp1 = sc_edge(pk_p, as1.reshape(NP), ad1.reshape(NP), mx1, xp1)
    return _final(o1, wp1, b1.reshape(1, H), mlp_W1,
                  mlp_b1.reshape(1, H), mlp_W2, mlp_b2.reshape(1, C),
                  batch_p)


# speedup vs baseline: 1.1878x; 1.1878x over previous
import functools

import jax
import jax.numpy as jnp
from jax import lax
from jax.experimental import pallas as pl
from jax.experimental.pallas import tpu as pltpu
from jax.experimental.pallas import tpu_sc as plsc

N = 10000
NP = 10240
NP2 = NP // 4
D = 128
H = 128
C = 10
G = 64
NC = 2
NS = 16
NW = NC * NS
CH = 128
BR = 1024
SLOPE = 0.2
DOT = (((1,), (1,)), ((), ()))
PREC = lax.Precision.HIGHEST


def _leaky(t):
    return jnp.maximum(t, SLOPE * t)


def _elu(t):
    return jnp.where(t > 0, t, jnp.exp(t) - 1.0)


def _xform_tail(h, w_ref, atts_ref, attd_ref, xp_ref, as_ref, ad_ref,
                mx_ref, acc_ref):
    i = pl.program_id(0)
    xp = lax.dot_general(h, w_ref[...], DOT, precision=PREC,
                         preferred_element_type=jnp.float32)
    xp_ref[...] = xp
    a_s = lax.dot_general(xp, atts_ref[...], (((1,), (0,)), ((), ())),
                          precision=PREC, preferred_element_type=jnp.float32)
    a_d = lax.dot_general(xp, attd_ref[...], (((1,), (0,)), ((), ())),
                          precision=PREC, preferred_element_type=jnp.float32)
    as_ref[...] = a_s
    ad_ref[...] = a_d
    m = jnp.max(a_s)
    prev = jnp.where(i == 0, -jnp.inf, acc_ref[0, 0])
    cur = jnp.maximum(prev, m)
    acc_ref[0, 0] = cur

    @pl.when(i == pl.num_programs(0) - 1)
    def _():
        mx_ref[...] = jnp.full((1, 16), cur, jnp.float32)


def _xform_a_body(x_ref, w_ref, atts_ref, attd_ref,
                  xp_ref, as_ref, ad_ref, mx_ref, acc_ref):
    _xform_tail(x_ref[...], w_ref, atts_ref, attd_ref,
                xp_ref, as_ref, ad_ref, mx_ref, acc_ref)


def _combine(p_ref, wp_ref, b_ref):
    wsum = jnp.sum(wp_ref[...], axis=0) + 1e-16
    num = jnp.sum(p_ref[...], axis=0)
    return _elu(num / wsum[:, None] + b_ref[...])


def _xform_b_body(p_ref, wp_ref, b_ref, w_ref, atts_ref, attd_ref,
                  xp_ref, as_ref, ad_ref, mx_ref, acc_ref):
    h = _combine(p_ref, wp_ref, b_ref)
    _xform_tail(h, w_ref, atts_ref, attd_ref,
                xp_ref, as_ref, ad_ref, mx_ref, acc_ref)


def _final_body(p_ref, wp_ref, b_ref, mw1_ref, mb1_ref,
                mw2_ref, mb2_ref, batch_ref, out_ref):
    i = pl.program_id(0)
    h = _combine(p_ref, wp_ref, b_ref)
    z = lax.dot_general(h, mw1_ref[...], DOT, precision=PREC,
                        preferred_element_type=jnp.float32) + mb1_ref[...]
    z = jnp.maximum(z, 0.0)
    z = lax.dot_general(z, mw2_ref[...], DOT, precision=PREC,
                        preferred_element_type=jnp.float32) + mb2_ref[...]
    seg = lax.broadcasted_iota(jnp.int32, (BR, G), 1)
    oh = (batch_ref[...] == seg).astype(jnp.float32)
    contrib = lax.dot_general(oh, z, (((0,), (0,)), ((), ())),
                              precision=PREC,
                              preferred_element_type=jnp.float32)

    @pl.when(i == 0)
    def _():
        out_ref[...] = jnp.zeros_like(out_ref)

    out_ref[...] = out_ref[...] + contrib


def _row_spec(shape=(BR, H)):
    return pl.BlockSpec(shape, lambda i: (i, 0))


def _full(shape):
    return pl.BlockSpec(shape, lambda i: tuple(0 for _ in shape))


_PART_SPEC = pl.BlockSpec((NC, BR, H), lambda i: (0, i, 0))
_WP_SPEC = pl.BlockSpec((NW, BR), lambda i: (0, i))

_XFORM_OUTS = (
    jax.ShapeDtypeStruct((NP, H), jnp.float32),
    jax.ShapeDtypeStruct((NP, 1), jnp.float32),
    jax.ShapeDtypeStruct((NP, 1), jnp.float32),
    jax.ShapeDtypeStruct((1, 16), jnp.float32),
)
_XFORM_OUT_SPECS = [
    _row_spec(),
    _row_spec((BR, 1)),
    _row_spec((BR, 1)),
    _full((1, 16)),
]

_xform_a = pl.pallas_call(
    _xform_a_body,
    grid=(NP // BR,),
    in_specs=[
        _row_spec(),
        _full((H, D)),
        _full((D, 1)),
        _full((D, 1)),
    ],
    out_specs=_XFORM_OUT_SPECS,
    out_shape=_XFORM_OUTS,
    scratch_shapes=[pltpu.SMEM((1, 1), jnp.float32)],
)

_xform_b = pl.pallas_call(
    _xform_b_body,
    grid=(NP // BR,),
    in_specs=[
        _PART_SPEC,
        _WP_SPEC,
        _full((1, H)),
        _full((H, H)),
        _full((H, 1)),
        _full((H, 1)),
    ],
    out_specs=_XFORM_OUT_SPECS,
    out_shape=_XFORM_OUTS,
    scratch_shapes=[pltpu.SMEM((1, 1), jnp.float32)],
)

_final = pl.pallas_call(
    _final_body,
    grid=(NP // BR,),
    in_specs=[
        _PART_SPEC,
        _WP_SPEC,
        _full((1, H)),
        _full((H, H)),
        _full((1, H)),
        _full((C, H)),
        _full((1, C)),
        _row_spec((BR, 1)),
    ],
    out_specs=_full((G, C)),
    out_shape=jax.ShapeDtypeStruct((G, C), jnp.float32),
)


def _make_sc_edge(ncz, e_total):
    epw = ncz * CH
    nrp = NP2 // NS
    mesh = plsc.VectorSubcoreMesh(core_axis_name="c", subcore_axis_name="s",
                                  num_cores=NC, num_subcores=NS)

    @functools.partial(
        pl.kernel,
        out_type=(
            pltpu.MemorySpace.HBM((NC, NP, H), jnp.float32),
            pltpu.MemorySpace.HBM((NW, NP), jnp.float32),
        ),
        mesh=mesh,
        scratch_types=[
            pltpu.VMEM((ncz, CH), jnp.int32),
            pltpu.VMEM((ncz, CH), jnp.int32),
            pltpu.VMEM((3, CH), jnp.int32),
            pltpu.VMEM((3, CH), jnp.float32),
            pltpu.VMEM((NP,), jnp.float32),
            pltpu.VMEM((NP,), jnp.float32),
            pltpu.VMEM((1, 16), jnp.float32),
            pltpu.VMEM((NP,), jnp.float32),
            pltpu.VMEM((CH, H), jnp.float32),
            pltpu.VMEM((CH, H), jnp.float32),
            pltpu.VMEM((CH, H), jnp.float32),
            pltpu.VMEM_SHARED((NP2, H), jnp.float32),
            pltpu.SemaphoreType.DMA,
            pltpu.SemaphoreType.DMA,
            pltpu.SemaphoreType.DMA,
            pltpu.SemaphoreType.DMA,
            pltpu.SemaphoreType.DMA,
            pltpu.SemaphoreType.DMA,
        ],
        compiler_params=pltpu.CompilerParams(needs_layout_passes=False),
    )
    def sc_edge(src_hbm, dst_hbm, as_hbm, ad_hbm, mx_hbm, xp_hbm,
                out_hbm, wp_hbm,
                src_v, dst_v, dstp_v, exp_v, as_v, ad_v, mx_v, w_v,
                rows0_v, rows1_v, rows2_v, out_sp, g0, g1, g2, s0, s1, s2):
        rows = [rows0_v, rows1_v, rows2_v]
        gsem = [g0, g1, g2]
        ssem = [s0, s1, s2]
        rows_v = rows0_v
        cid = lax.axis_index("c")
        sid = lax.axis_index("s")
        wid = cid * NS + sid
        pltpu.sync_copy(src_hbm.at[wid], src_v)
        pltpu.sync_copy(dst_hbm.at[wid], dst_v)
        pltpu.sync_copy(as_hbm, as_v)
        pltpu.sync_copy(ad_hbm, ad_v)
        pltpu.sync_copy(mx_hbm, mx_v)

        zero16 = jnp.zeros((16,), jnp.float32)

        def _zw(i, carry):
            w_v[pl.ds(i * 16, 16)] = zero16
            return carry

        lax.fori_loop(0, NP // 16, _zw, 0)

        def _zr(i, carry):
            for j in range(H // 16):
                rows_v[i, pl.ds(j * 16, 16)] = zero16
            return carry

        def _zero_my_outsp_slice():
            for k in range(nrp // CH):
                pltpu.sync_copy(rows_v,
                                out_sp.at[pl.ds(sid * nrp + k * CH, CH)])
            rem = nrp % CH
            if rem:
                pltpu.sync_copy(
                    rows_v.at[pl.ds(0, rem)],
                    out_sp.at[pl.ds(sid * nrp + (nrp // CH) * CH, rem)])

        lax.fori_loop(0, CH, _zr, 0)
        _zero_my_outsp_slice()
        plsc.subcore_barrier()

        mxv = mx_v[0, :]
        ebase = wid * epw
        iota16 = lax.iota(jnp.int32, 16)

        def _stage(c, k, lo, accw):
            for g in range(CH // 16):
                sl = pl.ds(g * 16, 16)
                s16 = src_v[c, sl]
                d16 = dst_v[c, sl]
                a1 = plsc.load_gather(as_v, [s16])
                a2 = plsc.load_gather(ad_v, [d16])
                alpha = _leaky(a1 + a2)
                mb = _leaky(mxv + a2)
                ex16 = jnp.exp(alpha - mb)
                eid = ebase + c * CH + g * 16 + iota16
                ex16 = jnp.where(eid < e_total, ex16, 0.0)
                if accw:
                    plsc.addupdate_scatter(w_v, [d16], ex16)
                rel = d16 - lo
                inr = (rel >= 0) & (rel < NP2)
                exp_v[k, sl] = jnp.where(inr, ex16, 0.0)
                dstp_v[k, sl] = jnp.clip(rel, 0, NP2 - 1)

        def _gather_start(c, k):
            pltpu.async_copy(xp_hbm.at[src_v.at[c]], rows[k], gsem[k])

        def _gather_wait(c, k):
            pltpu.make_async_copy(xp_hbm.at[src_v.at[c]], rows[k],
                                  gsem[k]).wait()

        def _scat_start(k):
            pltpu.async_copy(rows[k], out_sp.at[dstp_v.at[k]], ssem[k],
                             add=True)

        def _scat_wait(k):
            pltpu.make_async_copy(rows[k], out_sp.at[dstp_v.at[k]],
                                  ssem[k]).wait()

        def _scale(k):
            def body(g2, carry2):
                ex16 = exp_v[k, pl.ds(g2 * 16, 16)]
                for l in range(16):
                    sc = ex16[l]
                    for j in range(H // 16):
                        sl2 = pl.ds(j * 16, 16)
                        i = g2 * 16 + l
                        rows[k][i, sl2] = rows[k][i, sl2] * sc
                return carry2

            lax.fori_loop(0, CH // 16, body, 0)

        def _flush(row0):
            pltpu.sync_copy(out_sp.at[pl.ds(sid * nrp, nrp)],
                            out_hbm.at[cid, pl.ds(row0 + sid * nrp, nrp)])

        for p in range(NP // NP2):
            lo = p * NP2
            if p:
                plsc.subcore_barrier()
                lax.fori_loop(0, CH, _zr, 0)
                _zero_my_outsp_slice()
                plsc.subcore_barrier()

            _gather_start(0, 0)

            def _tri(i3, carry, _lo=lo, _accw=(p == 0)):
                for k in range(3):
                    c = 3 * i3 + k
                    nxt = c + 1
                    kn = (k + 1) % 3

                    @pl.when(nxt < ncz)
                    def _():
                        @pl.when(c >= 2)
                        def _():
                            _scat_wait(kn)

                        _gather_start(nxt, kn)

                    _stage(c, k, _lo, _accw)
                    _gather_wait(c, k)
                    _scale(k)
                    _scat_start(k)
                return carry

            lax.fori_loop(0, ncz // 3, _tri, 0)
            for k in range(3):
                _scat_wait(k)
            plsc.subcore_barrier()
            _flush(lo)
        pltpu.sync_copy(w_v, wp_hbm.at[wid])

    return sc_edge


def kernel(x, edge_index, batch, W0, att_src0, att_dst0, b0,
           W1, att_src1, att_dst1, b1, mlp_W1, mlp_b1, mlp_W2, mlp_b2):
    n = x.shape[0]
    e = edge_index.shape[1]
    e_total = e + n
    ncz = -(-e_total // (NW * CH))
    ncz += (-ncz) % 3
    e_pad = NW * ncz * CH

    loops = jnp.arange(n, dtype=edge_index.dtype)
    src = jnp.concatenate([edge_index[0], loops])
    dst = jnp.concatenate([edge_index[1], loops])
    src_p = jnp.pad(src, (0, e_pad - e_total)).reshape(NW, ncz, CH)
    dst_p = jnp.pad(dst, (0, e_pad - e_total)).reshape(NW, ncz, CH)
    x_p = jnp.pad(x, ((0, NP - n), (0, 0)))
    batch_p = jnp.pad(batch, (0, NP - n), constant_values=G).reshape(NP, 1)

    sc_edge = _make_sc_edge(ncz, e_total)

    xp0, as0, ad0, mx0 = _xform_a(
        x_p, W0, att_src0.reshape(D, 1), att_dst0.reshape(D, 1))
    o0, wp0 = sc_edge(src_p, dst_p, as0.reshape(NP), ad0.reshape(NP),
                      mx0, xp0)
    xp1, as1, ad1, mx1 = _xform_b(
        o0, wp0, b0.reshape(1, H), W1,
        att_src1.reshape(H, 1), att_dst1.reshape(H, 1))
    o1, wp1 = sc_edge(src_p, dst_p, as1.reshape(NP), ad1.reshape(NP),
                      mx1, xp1)
    return _final(o1, wp1, b1.reshape(1, H), mlp_W1,
                  mlp_b1.reshape(1, H), mlp_W2, mlp_b2.reshape(1, C),
                  batch_p)
